# baseline (speedup 1.0000x reference)
import functools

import jax
import jax.numpy as jnp
from jax.experimental import pallas as pl
from jax.experimental.pallas import tpu as pltpu

N_LAYERS = 3
N_CHUNKS = 8


def _gcn_kernel(a_ref, c_ref, d_ref, w_ref, circ_out, dis_out, score_out,
                ab_s, egocT_s, degc_s, degd_s, yd1_s, tmpT_s, accdT_s):
    i = pl.program_id(0)
    CH, D = a_ref.shape
    C = CH * N_CHUNKS
    L = c_ref.shape[1]

    @pl.when(i == 0)
    def _init():
        egocT_s[:] = jnp.transpose(c_ref[:])
        degd_s[:] = jnp.zeros_like(degd_s)
        yd1_s[:] = jnp.zeros_like(yd1_s)

    @pl.when(i < N_CHUNKS)
    def _stream_chunk():
        a_ch = a_ref[:]
        abch = a_ch.astype(jnp.bfloat16)
        ab_s[pl.ds(i * CH, CH), :] = abch
        rs = jax.lax.dot_general(
            jnp.ones((1, D), jnp.float32), a_ch, (((1,), (1,)), ((), ())),
            preferred_element_type=jnp.float32)
        degc_s[:, pl.ds(i * CH, CH)] = rs
        degd_s[:] += jax.lax.dot_general(
            jnp.ones((1, CH), jnp.float32), a_ch, (((1,), (0,)), ((), ())),
            preferred_element_type=jnp.float32)
        dc_ch = jnp.where(rs > 0, jax.lax.rsqrt(rs), 0.0)
        zcT_ch = dc_ch * egocT_s[:, pl.ds(i * CH, CH)]
        yd1_s[:] += jax.lax.dot_general(
            zcT_ch.astype(jnp.bfloat16), abch, (((1,), (0,)), ((), ())),
            preferred_element_type=jnp.float32)

    @pl.when(i == N_CHUNKS)
    def _propagate():
        ab = ab_s[:]
        ego_cT = egocT_s[:]
        ego_dT = jnp.transpose(d_ref[:])
        deg_c = degc_s[:]
        deg_d = degd_s[:]
        dc = jnp.where(deg_c > 0, jax.lax.rsqrt(deg_c), 0.0)
        dd = jnp.where(deg_d > 0, jax.lax.rsqrt(deg_d), 0.0)

        def cos_weight(yT, egoT):
            num = jnp.sum(yT * egoT, axis=0, keepdims=True)
            ny = jnp.sqrt(jnp.sum(yT * yT, axis=0, keepdims=True))
            ne = jnp.sqrt(jnp.sum(egoT * egoT, axis=0, keepdims=True))
            return num / jnp.maximum(ny * ne, 1e-8)

        ycT = dc * jax.lax.dot_general(
            (dd * ego_dT).astype(jnp.bfloat16), ab, (((1,), (1,)), ((), ())),
            preferred_element_type=jnp.float32)
        ydT = dd * yd1_s[:]
        xcT = cos_weight(ycT, ego_cT) * ycT
        xdT = cos_weight(ydT, ego_dT) * ydT
        acc_cT = xcT
        acc_dT = xdT
        for _ in range(N_LAYERS - 1):
            ycT = dc * jax.lax.dot_general(
                (dd * xdT).astype(jnp.bfloat16), ab, (((1,), (1,)), ((), ())),
                preferred_element_type=jnp.float32)
            ydT = dd * jax.lax.dot_general(
                (dc * xcT).astype(jnp.bfloat16), ab, (((1,), (0,)), ((), ())),
                preferred_element_type=jnp.float32)
            xcT = cos_weight(ycT, ego_cT) * ycT
            xdT = cos_weight(ydT, ego_dT) * ydT
            acc_cT = acc_cT + xcT
            acc_dT = acc_dT + xdT

        circ_out[:] = jnp.transpose(acc_cT)
        dis_out[:] = jnp.transpose(acc_dT)
        tmpT_s[:] = jax.lax.dot_general(
            w_ref[:], acc_cT, (((0,), (0,)), ((), ())),
            preferred_element_type=jnp.float32).astype(jnp.bfloat16)
        accdT_s[:] = acc_dT.astype(jnp.bfloat16)

    @pl.when(i >= N_CHUNKS)
    def _score_tile():
        t = i - N_CHUNKS
        score_out[:] = jax.lax.dot_general(
            tmpT_s[:, pl.ds(t * CH, CH)], accdT_s[:], (((0,), (0,)), ((), ())),
            preferred_element_type=jnp.float32)


@functools.partial(jax.jit)
def kernel(A, circ_emb, dis_emb, re_CD):
    C, D = A.shape
    L = circ_emb.shape[1]
    CH = C // N_CHUNKS
    out_shapes = (
        jax.ShapeDtypeStruct((C, L), jnp.float32),
        jax.ShapeDtypeStruct((D, L), jnp.float32),
        jax.ShapeDtypeStruct((C, D), jnp.float32),
    )
    return pl.pallas_call(
        _gcn_kernel,
        grid=(2 * N_CHUNKS,),
        in_specs=[
            pl.BlockSpec((CH, D), lambda i: (jnp.minimum(i, N_CHUNKS - 1), 0)),
            pl.BlockSpec((C, L), lambda i: (0, 0)),
            pl.BlockSpec((D, L), lambda i: (0, 0)),
            pl.BlockSpec((L, L), lambda i: (0, 0)),
        ],
        out_specs=(
            pl.BlockSpec((C, L), lambda i: (0, 0)),
            pl.BlockSpec((D, L), lambda i: (0, 0)),
            pl.BlockSpec((CH, D), lambda i: (jnp.maximum(i - N_CHUNKS, 0), 0)),
        ),
        scratch_shapes=[
            pltpu.VMEM((C, D), jnp.bfloat16),
            pltpu.VMEM((L, C), jnp.float32),
            pltpu.VMEM((1, C), jnp.float32),
            pltpu.VMEM((1, D), jnp.float32),
            pltpu.VMEM((L, D), jnp.float32),
            pltpu.VMEM((L, C), jnp.bfloat16),
            pltpu.VMEM((L, D), jnp.bfloat16),
        ],
        out_shape=out_shapes,
        compiler_params=pltpu.CompilerParams(
            vmem_limit_bytes=100 * 1024 * 1024,
        ),
    )(A, circ_emb, dis_emb, re_CD)


# speedup vs baseline: 1.0383x; 1.0383x over previous
import functools

import jax
import jax.numpy as jnp
from jax.experimental import pallas as pl
from jax.experimental.pallas import tpu as pltpu

N_LAYERS = 3


def _gcn_kernel(a_ref, c_ref, d_ref, w_ref, circ_out, dis_out, score_out):
    a = a_ref[:]
    C, D = a.shape
    ego_cT = jnp.transpose(c_ref[:])
    ego_dT = jnp.transpose(d_ref[:])

    deg_c = jax.lax.dot_general(
        jnp.ones((1, D), jnp.float32), a, (((1,), (1,)), ((), ())),
        preferred_element_type=jnp.float32)
    deg_d = jax.lax.dot_general(
        jnp.ones((1, C), jnp.float32), a, (((1,), (0,)), ((), ())),
        preferred_element_type=jnp.float32)
    dc = jnp.where(deg_c > 0, jax.lax.rsqrt(deg_c), 0.0)
    dd = jnp.where(deg_d > 0, jax.lax.rsqrt(deg_d), 0.0)

    def cos_weight(yT, egoT):
        num = jnp.sum(yT * egoT, axis=0, keepdims=True)
        ny = jnp.sqrt(jnp.sum(yT * yT, axis=0, keepdims=True))
        ne = jnp.sqrt(jnp.sum(egoT * egoT, axis=0, keepdims=True))
        return num / jnp.maximum(ny * ne, 1e-8)

    xcT, xdT = ego_cT, ego_dT
    acc_cT = jnp.zeros_like(ego_cT)
    acc_dT = jnp.zeros_like(ego_dT)
    for _ in range(N_LAYERS):
        ycT = dc * jax.lax.dot_general(
            dd * xdT, a, (((1,), (1,)), ((), ())),
            preferred_element_type=jnp.float32)
        ydT = dd * jax.lax.dot_general(
            dc * xcT, a, (((1,), (0,)), ((), ())),
            preferred_element_type=jnp.float32)
        xcT = cos_weight(ycT, ego_cT) * ycT
        xdT = cos_weight(ydT, ego_dT) * ydT
        acc_cT = acc_cT + xcT
        acc_dT = acc_dT + xdT

    circ_out[:] = jnp.transpose(acc_cT)
    dis_out[:] = jnp.transpose(acc_dT)
    tmpT = jax.lax.dot_general(
        w_ref[:], acc_cT, (((0,), (0,)), ((), ())),
        preferred_element_type=jnp.float32)
    score_out[:] = jax.lax.dot_general(
        tmpT, acc_dT, (((0,), (0,)), ((), ())),
        preferred_element_type=jnp.float32)


@functools.partial(jax.jit)
def kernel(A, circ_emb, dis_emb, re_CD):
    C, D = A.shape
    L = circ_emb.shape[1]
    out_shapes = (
        jax.ShapeDtypeStruct((C, L), jnp.float32),
        jax.ShapeDtypeStruct((D, L), jnp.float32),
        jax.ShapeDtypeStruct((C, D), jnp.float32),
    )
    return pl.pallas_call(
        _gcn_kernel,
        out_shape=out_shapes,
        compiler_params=pltpu.CompilerParams(
            vmem_limit_bytes=100 * 1024 * 1024,
        ),
    )(A, circ_emb, dis_emb, re_CD)


# speedup vs baseline: 1.0401x; 1.0017x over previous
import functools

import jax
import jax.numpy as jnp
from jax.experimental import pallas as pl
from jax.experimental.pallas import tpu as pltpu

N_LAYERS = 3


def _gcn_kernel(a_ref, c_ref, d_ref, w_ref, circ_out, dis_out, score_out):
    a = a_ref[:]
    C, D = a.shape
    ego_cT = jnp.transpose(c_ref[:])
    ego_dT = jnp.transpose(d_ref[:])

    deg_c = jax.lax.dot_general(
        jnp.ones((1, D), jnp.float32), a, (((1,), (1,)), ((), ())),
        preferred_element_type=jnp.float32)
    deg_d = jax.lax.dot_general(
        jnp.ones((1, C), jnp.float32), a, (((1,), (0,)), ((), ())),
        preferred_element_type=jnp.float32)
    dc = jnp.where(deg_c > 0, jax.lax.rsqrt(deg_c), 0.0)
    dd = jnp.where(deg_d > 0, jax.lax.rsqrt(deg_d), 0.0)

    ne_c = jnp.sqrt(jnp.sum(ego_cT * ego_cT, axis=0, keepdims=True))
    ne_d = jnp.sqrt(jnp.sum(ego_dT * ego_dT, axis=0, keepdims=True))

    def cos_weight(yT, egoT, ne):
        num = jnp.sum(yT * egoT, axis=0, keepdims=True)
        ny = jnp.sqrt(jnp.sum(yT * yT, axis=0, keepdims=True))
        return num / jnp.maximum(ny * ne, 1e-8)

    xcT, xdT = ego_cT, ego_dT
    acc_cT = None
    acc_dT = None
    for _ in range(N_LAYERS):
        ycT = dc * jax.lax.dot_general(
            dd * xdT, a, (((1,), (1,)), ((), ())),
            preferred_element_type=jnp.float32)
        ydT = dd * jax.lax.dot_general(
            dc * xcT, a, (((1,), (0,)), ((), ())),
            preferred_element_type=jnp.float32)
        xcT = cos_weight(ycT, ego_cT, ne_c) * ycT
        xdT = cos_weight(ydT, ego_dT, ne_d) * ydT
        acc_cT = xcT if acc_cT is None else acc_cT + xcT
        acc_dT = xdT if acc_dT is None else acc_dT + xdT

    circ_out[:] = jnp.transpose(acc_cT)
    dis_out[:] = jnp.transpose(acc_dT)
    tmpT = jax.lax.dot_general(
        w_ref[:], acc_cT, (((0,), (0,)), ((), ())),
        preferred_element_type=jnp.float32)
    score_out[:] = jax.lax.dot_general(
        tmpT, acc_dT, (((0,), (0,)), ((), ())),
        preferred_element_type=jnp.float32)


@functools.partial(jax.jit)
def kernel(A, circ_emb, dis_emb, re_CD):
    C, D = A.shape
    L = circ_emb.shape[1]
    out_shapes = (
        jax.ShapeDtypeStruct((C, L), jnp.float32),
        jax.ShapeDtypeStruct((D, L), jnp.float32),
        jax.ShapeDtypeStruct((C, D), jnp.float32),
    )
    return pl.pallas_call(
        _gcn_kernel,
        out_shape=out_shapes,
        compiler_params=pltpu.CompilerParams(
            vmem_limit_bytes=100 * 1024 * 1024,
        ),
    )(A, circ_emb, dis_emb, re_CD)
